# baseline (speedup 1.0000x reference)
import functools

import jax
import jax.numpy as jnp
from jax import lax
from jax.experimental import pallas as pl
from jax.experimental.pallas import tpu as pltpu
from jax.experimental.pallas import tpu_sc as plsc

_IDX_CHUNK = 128


def _make_sc_gather(num_users, num_items, emb, batch, nc, ns):
    nw = nc * ns
    b_per_w = batch // nw
    n_chunks = b_per_w // _IDX_CHUNK
    mesh = plsc.VectorSubcoreMesh(core_axis_name="c", subcore_axis_name="s")

    @functools.partial(
        pl.kernel,
        mesh=mesh,
        compiler_params=pltpu.CompilerParams(use_tc_tiling_on_sc=False),
        out_type=[
            jax.ShapeDtypeStruct((batch, emb), jnp.float32),
            jax.ShapeDtypeStruct((batch, emb), jnp.float32),
        ],
        scratch_types=[
            pltpu.VMEM((n_chunks, _IDX_CHUNK), jnp.int32),
            pltpu.VMEM((b_per_w, emb), jnp.float32),
            pltpu.VMEM((n_chunks, _IDX_CHUNK), jnp.int32),
            pltpu.VMEM((b_per_w, emb), jnp.float32),
            pltpu.SemaphoreType.DMA,
            pltpu.SemaphoreType.DMA,
        ],
    )
    def gather_k(uid_hbm, utab_hbm, iid_hbm, itab_hbm, ue_hbm, ie_hbm,
                 uidx_v, urows_v, iidx_v, irows_v, usem, isem):
        wid = lax.axis_index("s") * nc + lax.axis_index("c")
        base = wid * b_per_w
        for j in range(n_chunks):
            pltpu.sync_copy(uid_hbm.at[pl.ds(base + j * _IDX_CHUNK, _IDX_CHUNK)],
                            uidx_v.at[j])
            pltpu.sync_copy(iid_hbm.at[pl.ds(base + j * _IDX_CHUNK, _IDX_CHUNK)],
                            iidx_v.at[j])
        copies = []
        for j in range(n_chunks):
            copies.append(pltpu.async_copy(
                utab_hbm.at[uidx_v.at[j]],
                urows_v.at[pl.ds(j * _IDX_CHUNK, _IDX_CHUNK)], usem))
            copies.append(pltpu.async_copy(
                itab_hbm.at[iidx_v.at[j]],
                irows_v.at[pl.ds(j * _IDX_CHUNK, _IDX_CHUNK)], isem))
        for c in copies:
            c.wait()
        pltpu.sync_copy(urows_v, ue_hbm.at[pl.ds(base, b_per_w)])
        pltpu.sync_copy(irows_v, ie_hbm.at[pl.ds(base, b_per_w)])

    return gather_k


def _mlp_body(ue_ref, ie_ref, w1u_ref, w1i_ref, b1_ref, w2_ref, b2_ref,
              w3_ref, b3_ref, out_ref):
    cdims = (((1,), (1,)), ((), ()))
    h1 = lax.dot_general(ue_ref[...], w1u_ref[...], cdims,
                         preferred_element_type=jnp.float32)
    h1 = h1 + lax.dot_general(ie_ref[...], w1i_ref[...], cdims,
                              preferred_element_type=jnp.float32)
    h1 = jnp.maximum(h1 + b1_ref[...], 0.0)
    h2 = lax.dot_general(h1, w2_ref[...], cdims,
                         preferred_element_type=jnp.float32)
    h2 = jnp.maximum(h2 + b2_ref[...], 0.0)
    logit = jnp.sum(h2 * w3_ref[...], axis=1, keepdims=True) + b3_ref[...]
    out_ref[...] = jax.nn.sigmoid(logit)


def _mlp_tc(ue, ie, W1u, W1i, b1, W2, b2, W3, b3, block_b):
    batch = ue.shape[0]
    grid = (batch // block_b,)
    full = lambda shape: pl.BlockSpec(shape, lambda i: (0, 0))
    return pl.pallas_call(
        _mlp_body,
        grid=grid,
        in_specs=[
            pl.BlockSpec((block_b, ue.shape[1]), lambda i: (i, 0)),
            pl.BlockSpec((block_b, ie.shape[1]), lambda i: (i, 0)),
            full(W1u.shape),
            full(W1i.shape),
            full(b1.shape),
            full(W2.shape),
            full(b2.shape),
            full(W3.shape),
            full(b3.shape),
        ],
        out_specs=pl.BlockSpec((block_b, 1), lambda i: (i, 0)),
        out_shape=jax.ShapeDtypeStruct((batch, 1), jnp.float32),
    )(ue, ie, W1u, W1i, b1, W2, b2, W3, b3)


def kernel(user_ids, item_ids, user_table, item_table, W1, b1, W2, b2, W3, b3):
    batch = user_ids.shape[0]
    emb = user_table.shape[1]
    info = plsc.get_sparse_core_info()
    nc, ns = info.num_cores, info.num_subcores

    uid = user_ids.astype(jnp.int32)
    iid = item_ids.astype(jnp.int32)
    gather_k = _make_sc_gather(user_table.shape[0], item_table.shape[0],
                               emb, batch, nc, ns)
    ue, ie = gather_k(uid, user_table, iid, item_table)

    W1u = W1[:, :emb]
    W1i = W1[:, emb:]
    return _mlp_tc(ue, ie, W1u, W1i, b1.reshape(1, -1), W2,
                   b2.reshape(1, -1), W3, b3.reshape(1, 1), block_b=2048)
